# baseline (speedup 1.0000x reference)
import functools

import jax
import jax.numpy as jnp
from jax import lax
from jax.experimental import pallas as pl
from jax.experimental.pallas import tpu as pltpu
from jax.experimental.pallas import tpu_sc as plsc

NUM_BINS = 1000000
EMB_DIM = 16
BATCH = 16384

_NC = 2
_NS = 16
_L = 16
_NW = _NC * _NS
_BPW = BATCH // _NW
_SB = _L
_NSB = _BPW // _SB


def _build_kernel():
    mesh = plsc.VectorSubcoreMesh(core_axis_name="c", subcore_axis_name="s")

    @functools.partial(
        pl.kernel,
        mesh=mesh,
        out_type=jax.ShapeDtypeStruct((EMB_DIM, BATCH), jnp.float32),
        scratch_types=[
            pltpu.VMEM((_BPW,), jnp.int32),
            pltpu.VMEM((_BPW,), jnp.int32),
            pltpu.VMEM((2, _SB, EMB_DIM, 128), jnp.float32),
            pltpu.VMEM((EMB_DIM, _BPW), jnp.float32),
            pltpu.SemaphoreType.DMA,
            pltpu.SemaphoreType.DMA,
        ],
        compiler_params=pltpu.CompilerParams(needs_layout_passes=False),
    )
    def k(ids_hbm, tab_hbm, out_hbm, ids_v, bins_v, blk_v, vals_v, sem0, sem1):
        wid = lax.axis_index("s") * _NC + lax.axis_index("c")
        base = wid * _BPW
        pltpu.sync_copy(ids_hbm.at[pl.ds(base, _BPW)], ids_v)
        for i in range(_BPW // _L):
            x = plsc.bitcast(ids_v[pl.ds(i * _L, _L)], jnp.uint32)
            h = (x * jnp.uint32(2654435761)) ^ (x >> jnp.uint32(16))
            b = h % jnp.uint32(NUM_BINS)
            bins_v[pl.ds(i * _L, _L)] = plsc.bitcast(b, jnp.int32)

        sems = (sem0, sem1)
        lane_iota = lax.iota(jnp.int32, _L)

        def fire(s, par):
            sem = sems[par]
            vec = bins_v[pl.ds(pl.multiple_of(s * _SB, _SB), _SB)]
            for j in range(_SB):
                b = jnp.sum(jnp.where(lane_iota == j, vec, 0))
                c = pl.multiple_of((b >> 7) << 7, 128)
                pltpu.async_copy(
                    tab_hbm.at[:, pl.ds(c, 128)], blk_v.at[par, j], sem
                )

        def drain_process(s, par):
            for j in range(_SB):
                pltpu.make_async_copy(
                    tab_hbm.at[:, pl.ds(0, 128)], blk_v.at[par, j], sems[par]
                ).wait()
            vec = bins_v[pl.ds(pl.multiple_of(s * _SB, _SB), _SB)]
            for j in range(_SB):
                lane = jnp.sum(jnp.where(lane_iota == j, vec & 127, 0))
                col_idx = jnp.full((_L,), lane, jnp.int32)
                v = plsc.load_gather(blk_v.at[par, j], [lane_iota, col_idx])
                out_col = jnp.full((_L,), s * _SB + j, jnp.int32)
                plsc.store_scatter(vals_v, [lane_iota, out_col], v)

        fire(0, 0)

        def body(t, carry):
            fire(2 * t + 1, 1)
            drain_process(2 * t, 0)

            @pl.when(t + 1 < _NSB // 2)
            def _():
                fire(2 * t + 2, 0)

            drain_process(2 * t + 1, 1)
            return carry

        lax.fori_loop(0, _NSB // 2, body, 0)

        pltpu.sync_copy(vals_v, out_hbm.at[:, pl.ds(base, _BPW)])

    return k


_lookup = _build_kernel()


def kernel(inputs, table):
    ids = inputs.reshape(BATCH)
    out_t = _lookup(ids, table.T)
    return out_t.T


# speedup vs baseline: 1.1131x; 1.1131x over previous
import functools

import jax
import jax.numpy as jnp
from jax import lax
from jax.experimental import pallas as pl
from jax.experimental.pallas import tpu as pltpu
from jax.experimental.pallas import tpu_sc as plsc

NUM_BINS = 1000000
EMB_DIM = 16
BATCH = 16384

_NC = 2
_NS = 16
_L = 16
_NW = _NC * _NS
_BPW = BATCH // _NW
_SB = 8
_NSB = _BPW // _SB
_NBUF = 4


def _build_kernel():
    mesh = plsc.VectorSubcoreMesh(core_axis_name="c", subcore_axis_name="s")

    @functools.partial(
        pl.kernel,
        mesh=mesh,
        out_type=jax.ShapeDtypeStruct((EMB_DIM, BATCH), jnp.float32),
        scratch_types=[
            pltpu.VMEM((_BPW,), jnp.int32),
            pltpu.VMEM((_BPW,), jnp.int32),
            pltpu.VMEM((_NBUF, _SB, EMB_DIM, 128), jnp.float32),
            pltpu.VMEM((EMB_DIM, _BPW), jnp.float32),
            pltpu.SemaphoreType.DMA,
            pltpu.SemaphoreType.DMA,
            pltpu.SemaphoreType.DMA,
            pltpu.SemaphoreType.DMA,
        ],
        compiler_params=pltpu.CompilerParams(needs_layout_passes=False),
    )
    def k(ids_hbm, tab_hbm, out_hbm, ids_v, bins_v, blk_v, vals_v,
          sem0, sem1, sem2, sem3):
        wid = lax.axis_index("s") * _NC + lax.axis_index("c")
        base = wid * _BPW
        pltpu.sync_copy(ids_hbm.at[pl.ds(base, _BPW)], ids_v)
        for i in range(_BPW // _L):
            x = plsc.bitcast(ids_v[pl.ds(i * _L, _L)], jnp.uint32)
            h = (x * jnp.uint32(2654435761)) ^ (x >> jnp.uint32(16))
            b = h % jnp.uint32(NUM_BINS)
            bins_v[pl.ds(i * _L, _L)] = plsc.bitcast(b, jnp.int32)

        sems = (sem0, sem1, sem2, sem3)
        lane_iota = lax.iota(jnp.int32, _L)

        def fire(s, buf):
            sem = sems[buf]
            vec = bins_v[pl.ds(pl.multiple_of((s // 2) * _L, _L), _L)]
            off = (s % 2) * _SB
            for j in range(_SB):
                b = jnp.sum(jnp.where(lane_iota == j + off, vec, 0))
                c = pl.multiple_of((b >> 7) << 7, 128)
                pltpu.async_copy(
                    tab_hbm.at[:, pl.ds(c, 128)], blk_v.at[buf, j], sem
                )

        def drain_process(s, buf):
            for j in range(_SB):
                pltpu.make_async_copy(
                    tab_hbm.at[:, pl.ds(0, 128)], blk_v.at[buf, j], sems[buf]
                ).wait()
            vec = bins_v[pl.ds(pl.multiple_of((s // 2) * _L, _L), _L)]
            off = (s % 2) * _SB
            for j in range(_SB):
                lane = jnp.sum(jnp.where(lane_iota == j + off, vec & 127, 0))
                col_idx = jnp.full((_L,), lane, jnp.int32)
                v = plsc.load_gather(blk_v.at[buf, j], [lane_iota, col_idx])
                out_col = jnp.full((_L,), s * _SB + j, jnp.int32)
                plsc.store_scatter(vals_v, [lane_iota, out_col], v)

        for s in range(_NBUF - 1):
            fire(s, s)

        def body(t, carry):
            for u in range(_NBUF):
                s = _NBUF * t + u

                @pl.when(s + _NBUF - 1 < _NSB)
                def _():
                    fire(s + _NBUF - 1, (u + _NBUF - 1) % _NBUF)

                drain_process(s, u)
            return carry

        lax.fori_loop(0, _NSB // _NBUF, body, 0)

        pltpu.sync_copy(vals_v, out_hbm.at[:, pl.ds(base, _BPW)])

    return k


_lookup = _build_kernel()


def kernel(inputs, table):
    ids = inputs.reshape(BATCH)
    out_t = _lookup(ids, table.T)
    return out_t.T
